# baseline (speedup 1.0000x reference)
import functools

import jax
import jax.numpy as jnp
import numpy as np
from jax import lax
from jax.experimental import pallas as pl
from jax.experimental.pallas import tpu as pltpu
from jax.experimental.pallas import tpu_sc as plsc

_ROWS = 64
_COLS = 8192
_KEEP = 256
_LANES = 16
_VECS = _COLS // _LANES
_NC = 2
_NS = 16
_ROWS_PER_W = _ROWS // (_NC * _NS)
_UNROLL = 8

_INT_MIN = np.int32(-2147483648)


def _order_key(b):
    return jnp.where(b >= 0, b, jnp.bitwise_xor(jnp.bitwise_not(b), _INT_MIN))


def _hsum(v):
    s = v[0]
    for lane in range(1, _LANES):
        s = s + v[lane]
    return s


def _count3(cs):
    c1, c2, c3 = cs[0]
    for u in range(1, _UNROLL):
        c1 = c1 + cs[u][0]
        c2 = c2 + cs[u][1]
        c3 = c3 + cs[u][2]
    return _hsum(c1), _hsum(c2), _hsum(c3)


def _step(acc, sh, cnt1, cnt2, cnt3):
    d = (jnp.where(cnt1 >= _KEEP, np.int32(1), np.int32(0))
         + jnp.where(cnt2 >= _KEEP, np.int32(1), np.int32(0))
         + jnp.where(cnt3 >= _KEEP, np.int32(1), np.int32(0)))
    return acc + (d << sh)


def _kwta_body(in_hbm, out_hbm, row_v, key_v):
    wid = lax.axis_index("s") * _NC + lax.axis_index("c")
    zeros16 = jnp.zeros((_LANES,), jnp.int32)

    for r in range(_ROWS_PER_W):
        row = wid * _ROWS_PER_W + r
        base = row * _COLS
        pltpu.sync_copy(in_hbm.at[pl.ds(base, _COLS)], row_v)

        t1 = _INT_MIN + (np.int32(1) << np.int32(30))
        t2 = _INT_MIN + (np.int32(2) << np.int32(30))
        t3 = _INT_MIN + (np.int32(3) << np.int32(30))

        def map_body(j, cs):
            base16 = j * (_LANES * _UNROLL)
            out = []
            for u in range(_UNROLL):
                x16 = row_v[pl.ds(base16 + u * _LANES, _LANES)]
                k16 = _order_key(lax.bitcast_convert_type(x16, jnp.int32))
                key_v[pl.ds(base16 + u * _LANES, _LANES)] = k16
                c1, c2, c3 = cs[u]
                out.append((
                    c1 + jnp.where(k16 >= t1, np.int32(1), np.int32(0)),
                    c2 + jnp.where(k16 >= t2, np.int32(1), np.int32(0)),
                    c3 + jnp.where(k16 >= t3, np.int32(1), np.int32(0)),
                ))
            return tuple(out)

        cs = lax.fori_loop(
            0, _VECS // _UNROLL, map_body,
            tuple((zeros16, zeros16, zeros16) for _ in range(_UNROLL)))
        cnt1, cnt2, cnt3 = _count3(cs)
        acc = _step(_INT_MIN, np.int32(30), cnt1, cnt2, cnt3)

        def bit_body(s, acc):
            sh = np.int32(28) - np.int32(2) * s
            t1 = acc + (np.int32(1) << sh)
            t2 = acc + (np.int32(2) << sh)
            t3 = acc + (np.int32(3) << sh)

            def cnt_body(j, cs):
                base16 = j * (_LANES * _UNROLL)
                out = []
                for u in range(_UNROLL):
                    k16 = key_v[pl.ds(base16 + u * _LANES, _LANES)]
                    c1, c2, c3 = cs[u]
                    out.append((
                        c1 + jnp.where(k16 >= t1, np.int32(1), np.int32(0)),
                        c2 + jnp.where(k16 >= t2, np.int32(1), np.int32(0)),
                        c3 + jnp.where(k16 >= t3, np.int32(1), np.int32(0)),
                    ))
                return tuple(out)

            cs = lax.fori_loop(
                0, _VECS // _UNROLL, cnt_body,
                tuple((zeros16, zeros16, zeros16) for _ in range(_UNROLL)))
            cnt1, cnt2, cnt3 = _count3(cs)
            return _step(acc, sh, cnt1, cnt2, cnt3)

        thr = lax.fori_loop(0, 15, bit_body, acc)

        def mask_body(j, carry):
            base16 = j * (_LANES * _UNROLL)
            for u in range(_UNROLL):
                x16 = row_v[pl.ds(base16 + u * _LANES, _LANES)]
                k16 = key_v[pl.ds(base16 + u * _LANES, _LANES)]
                row_v[pl.ds(base16 + u * _LANES, _LANES)] = jnp.where(
                    k16 >= thr, x16, np.float32(0.0))
            return carry

        lax.fori_loop(0, _VECS // _UNROLL, mask_body, np.int32(0))

        pltpu.sync_copy(row_v, out_hbm.at[pl.ds(base, _COLS)])


@functools.partial(jax.jit, static_argnums=())
def _kwta(flat):
    mesh = plsc.VectorSubcoreMesh(core_axis_name="c", subcore_axis_name="s")
    fn = functools.partial(
        pl.kernel,
        mesh=mesh,
        out_type=jax.ShapeDtypeStruct((_ROWS * _COLS,), jnp.float32),
        scratch_types=[
            pltpu.VMEM((_COLS,), jnp.float32),
            pltpu.VMEM((_COLS,), jnp.int32),
        ],
    )(_kwta_body)
    return fn(flat)


def kernel(inputs):
    out_flat = _kwta(inputs.reshape(-1))
    return out_flat.reshape(inputs.shape)


# speedup vs baseline: 1.3061x; 1.3061x over previous
import functools

import jax
import jax.numpy as jnp
import numpy as np
from jax import lax
from jax.experimental import pallas as pl
from jax.experimental.pallas import tpu as pltpu
from jax.experimental.pallas import tpu_sc as plsc

_ROWS = 64
_COLS = 8192
_KEEP = 256
_LANES = 16
_VECS = _COLS // _LANES
_NC = 2
_NS = 16
_ROWS_PER_W = _ROWS // (_NC * _NS)
_UNROLL = 4

_INT_MIN = np.int32(-2147483648)


def _unmap(c):
    b = jnp.where(c >= 0, c, jnp.bitwise_not(jnp.bitwise_xor(c, _INT_MIN)))
    return lax.bitcast_convert_type(b, jnp.float32)


def _hsum(v):
    s = v[0]
    for lane in range(1, _LANES):
        s = s + v[lane]
    return s


def _count3(cs):
    c1, c2, c3 = cs[0]
    for u in range(1, _UNROLL):
        c1 = c1 + cs[u][0]
        c2 = c2 + cs[u][1]
        c3 = c3 + cs[u][2]
    return _hsum(c1), _hsum(c2), _hsum(c3)


def _step(acc, sh, cnt1, cnt2, cnt3):
    d = (jnp.where(cnt1 >= _KEEP, np.int32(1), np.int32(0))
         + jnp.where(cnt2 >= _KEEP, np.int32(1), np.int32(0))
         + jnp.where(cnt3 >= _KEEP, np.int32(1), np.int32(0)))
    return acc + (d << sh)


def _kwta_body(in_hbm, out_hbm, row_v):
    wid = lax.axis_index("s") * _NC + lax.axis_index("c")
    zeros16 = jnp.zeros((_LANES,), jnp.int32)

    for r in range(_ROWS_PER_W):
        row = wid * _ROWS_PER_W + r
        base = row * _COLS
        pltpu.sync_copy(in_hbm.at[pl.ds(base, _COLS)], row_v)

        def bit_body(s, acc):
            sh = np.int32(30) - np.int32(2) * s
            tf1 = _unmap(acc + (np.int32(1) << sh))
            tf2 = _unmap(acc + (np.int32(2) << sh))
            tf3 = _unmap(acc + (np.int32(3) << sh))

            def cnt_body(j, cs):
                base16 = j * (_LANES * _UNROLL)
                out = []
                for u in range(_UNROLL):
                    x16 = row_v[pl.ds(base16 + u * _LANES, _LANES)]
                    c1, c2, c3 = cs[u]
                    out.append((
                        jnp.where(x16 >= tf1, c1 + np.int32(1), c1),
                        jnp.where(x16 >= tf2, c2 + np.int32(1), c2),
                        jnp.where(x16 >= tf3, c3 + np.int32(1), c3),
                    ))
                return tuple(out)

            cs = lax.fori_loop(
                0, _VECS // _UNROLL, cnt_body,
                tuple((zeros16, zeros16, zeros16) for _ in range(_UNROLL)))
            cnt1, cnt2, cnt3 = _count3(cs)
            return _step(acc, sh, cnt1, cnt2, cnt3)

        thr = _unmap(lax.fori_loop(0, 16, bit_body, _INT_MIN))

        def mask_body(j, carry):
            base16 = j * (_LANES * _UNROLL)
            for u in range(_UNROLL):
                x16 = row_v[pl.ds(base16 + u * _LANES, _LANES)]
                row_v[pl.ds(base16 + u * _LANES, _LANES)] = jnp.where(
                    x16 >= thr, x16, np.float32(0.0))
            return carry

        lax.fori_loop(0, _VECS // _UNROLL, mask_body, np.int32(0))

        pltpu.sync_copy(row_v, out_hbm.at[pl.ds(base, _COLS)])


@functools.partial(jax.jit, static_argnums=())
def _kwta(flat):
    mesh = plsc.VectorSubcoreMesh(core_axis_name="c", subcore_axis_name="s")
    fn = functools.partial(
        pl.kernel,
        mesh=mesh,
        out_type=jax.ShapeDtypeStruct((_ROWS * _COLS,), jnp.float32),
        scratch_types=[
            pltpu.VMEM((_COLS,), jnp.float32),
        ],
    )(_kwta_body)
    return fn(flat)


def kernel(inputs):
    out_flat = _kwta(inputs.reshape(-1))
    return out_flat.reshape(inputs.shape)


# speedup vs baseline: 1.5388x; 1.1782x over previous
import functools

import jax
import jax.numpy as jnp
import numpy as np
from jax import lax
from jax.experimental import pallas as pl
from jax.experimental.pallas import tpu as pltpu
from jax.experimental.pallas import tpu_sc as plsc

_ROWS = 64
_COLS = 8192
_KEEP = 256
_LANES = 16
_VECS = _COLS // _LANES
_NC = 2
_NS = 16
_ROWS_PER_W = _ROWS // (_NC * _NS)
_UNROLL = 8

_INT_MIN = np.int32(-2147483648)


def _order_key(b):
    return jnp.where(b >= 0, b, jnp.bitwise_xor(jnp.bitwise_not(b), _INT_MIN))


def _kwta_body(in_hbm, out_hbm, row_v, key_v):
    wid = lax.axis_index("s") * _NC + lax.axis_index("c")

    for r in range(_ROWS_PER_W):
        row = wid * _ROWS_PER_W + r
        base = row * _COLS
        pltpu.sync_copy(in_hbm.at[pl.ds(base, _COLS)], row_v)

        def map_body(j, carry):
            base16 = j * (_LANES * _UNROLL)
            for u in range(_UNROLL):
                x16 = row_v[pl.ds(base16 + u * _LANES, _LANES)]
                b16 = lax.bitcast_convert_type(x16, jnp.int32)
                key_v[pl.ds(base16 + u * _LANES, _LANES)] = _order_key(b16)
            return carry

        lax.fori_loop(0, _VECS // _UNROLL, map_body, np.int32(0))

        def bit_body(i, acc):
            trial = acc + (np.int32(1) << (np.int32(31) - i))

            def cnt_body(j, c16):
                base16 = j * (_LANES * _UNROLL)
                for u in range(_UNROLL):
                    k16 = key_v[pl.ds(base16 + u * _LANES, _LANES)]
                    c16 = jnp.where(k16 >= trial, c16 + np.int32(1), c16)
                return c16

            c16 = lax.fori_loop(0, _VECS // _UNROLL, cnt_body,
                                jnp.zeros((_LANES,), jnp.int32))
            cnt = c16[0]
            for lane in range(1, _LANES):
                cnt = cnt + c16[lane]
            return jnp.where(cnt >= _KEEP, trial, acc)

        thr = lax.fori_loop(0, 32, bit_body, _INT_MIN)

        def mask_body(j, carry):
            base16 = j * (_LANES * _UNROLL)
            for u in range(_UNROLL):
                x16 = row_v[pl.ds(base16 + u * _LANES, _LANES)]
                k16 = key_v[pl.ds(base16 + u * _LANES, _LANES)]
                row_v[pl.ds(base16 + u * _LANES, _LANES)] = jnp.where(
                    k16 >= thr, x16, np.float32(0.0))
            return carry

        lax.fori_loop(0, _VECS // _UNROLL, mask_body, np.int32(0))

        pltpu.sync_copy(row_v, out_hbm.at[pl.ds(base, _COLS)])


@functools.partial(jax.jit, static_argnums=())
def _kwta(flat):
    mesh = plsc.VectorSubcoreMesh(core_axis_name="c", subcore_axis_name="s")
    fn = functools.partial(
        pl.kernel,
        mesh=mesh,
        out_type=jax.ShapeDtypeStruct((_ROWS * _COLS,), jnp.float32),
        scratch_types=[
            pltpu.VMEM((_COLS,), jnp.float32),
            pltpu.VMEM((_COLS,), jnp.int32),
        ],
    )(_kwta_body)
    return fn(flat)


def kernel(inputs):
    out_flat = _kwta(inputs.reshape(-1))
    return out_flat.reshape(inputs.shape)


# speedup vs baseline: 1.5900x; 1.0333x over previous
import functools

import jax
import jax.numpy as jnp
import numpy as np
from jax import lax
from jax.experimental import pallas as pl
from jax.experimental.pallas import tpu as pltpu
from jax.experimental.pallas import tpu_sc as plsc

_ROWS = 64
_COLS = 8192
_KEEP = 256
_LANES = 16
_VECS = _COLS // _LANES
_NC = 2
_NS = 16
_ROWS_PER_W = _ROWS // (_NC * _NS)
_UNROLL = 4

_INT_MIN = np.int32(-2147483648)


def _order_key(b):
    return jnp.where(b >= 0, b, jnp.bitwise_xor(jnp.bitwise_not(b), _INT_MIN))


def _hsum(v):
    s = v[0]
    for lane in range(1, _LANES):
        s = s + v[lane]
    return s


def _kwta_body(in_hbm, out_hbm, row_v, key_v):
    wid = lax.axis_index("s") * _NC + lax.axis_index("c")
    zeros16 = jnp.zeros((_LANES,), jnp.int32)
    base = wid * (_ROWS_PER_W * _COLS)

    pltpu.sync_copy(in_hbm.at[pl.ds(base, _ROWS_PER_W * _COLS)], row_v)

    def map_body(j, carry):
        base16 = j * (_LANES * 2 * _UNROLL)
        for u in range(2 * _UNROLL):
            x16 = row_v[pl.ds(base16 + u * _LANES, _LANES)]
            key_v[pl.ds(base16 + u * _LANES, _LANES)] = _order_key(
                lax.bitcast_convert_type(x16, jnp.int32))
        return carry

    lax.fori_loop(0, 2 * _VECS // (2 * _UNROLL), map_body, np.int32(0))

    def bit_body(i, carry):
        acca, accb = carry
        bit = np.int32(1) << (np.int32(31) - i)
        ta = acca + bit
        tb = accb + bit

        def cnt_body(j, cc):
            ca, cb = cc
            base16 = j * (_LANES * _UNROLL)
            for u in range(_UNROLL):
                ka = key_v[pl.ds(base16 + u * _LANES, _LANES)]
                kb = key_v[pl.ds(_COLS + base16 + u * _LANES, _LANES)]
                ca = jnp.where(ka >= ta, ca + np.int32(1), ca)
                cb = jnp.where(kb >= tb, cb + np.int32(1), cb)
            return (ca, cb)

        ca, cb = lax.fori_loop(0, _VECS // _UNROLL, cnt_body,
                               (zeros16, zeros16))
        cnta = _hsum(ca)
        cntb = _hsum(cb)
        return (jnp.where(cnta >= _KEEP, ta, acca),
                jnp.where(cntb >= _KEEP, tb, accb))

    thra, thrb = lax.fori_loop(0, 32, bit_body, (_INT_MIN, _INT_MIN))

    def mask_body(j, carry):
        base16 = j * (_LANES * _UNROLL)
        for u in range(_UNROLL):
            for half, thr in ((0, thra), (_COLS, thrb)):
                x16 = row_v[pl.ds(half + base16 + u * _LANES, _LANES)]
                k16 = key_v[pl.ds(half + base16 + u * _LANES, _LANES)]
                row_v[pl.ds(half + base16 + u * _LANES, _LANES)] = jnp.where(
                    k16 >= thr, x16, np.float32(0.0))
        return carry

    lax.fori_loop(0, _VECS // _UNROLL, mask_body, np.int32(0))

    pltpu.sync_copy(row_v, out_hbm.at[pl.ds(base, _ROWS_PER_W * _COLS)])


@functools.partial(jax.jit, static_argnums=())
def _kwta(flat):
    mesh = plsc.VectorSubcoreMesh(core_axis_name="c", subcore_axis_name="s")
    fn = functools.partial(
        pl.kernel,
        mesh=mesh,
        out_type=jax.ShapeDtypeStruct((_ROWS * _COLS,), jnp.float32),
        scratch_types=[
            pltpu.VMEM((_ROWS_PER_W * _COLS,), jnp.float32),
            pltpu.VMEM((_ROWS_PER_W * _COLS,), jnp.int32),
        ],
    )(_kwta_body)
    return fn(flat)


def kernel(inputs):
    out_flat = _kwta(inputs.reshape(-1))
    return out_flat.reshape(inputs.shape)


# speedup vs baseline: 1.7231x; 1.0837x over previous
import functools

import jax
import jax.numpy as jnp
import numpy as np
from jax import lax
from jax.experimental import pallas as pl
from jax.experimental.pallas import tpu as pltpu
from jax.experimental.pallas import tpu_sc as plsc

_ROWS = 64
_COLS = 8192
_KEEP = 256
_LANES = 16
_VECS = _COLS // _LANES
_NC = 2
_NS = 16
_ROWS_PER_W = _ROWS // (_NC * _NS)
_UNROLL = 4

_INT_MIN = np.int32(-2147483648)


def _order_key(b):
    return jnp.where(b >= 0, b, jnp.bitwise_xor(jnp.bitwise_not(b), _INT_MIN))


def _hsum(v):
    s = v[0]
    for lane in range(1, _LANES):
        s = s + v[lane]
    return s


def _kwta_body(in_hbm, out_hbm, row_v, key_v):
    wid = lax.axis_index("s") * _NC + lax.axis_index("c")
    zeros16 = jnp.zeros((_LANES,), jnp.int32)
    base = wid * (_ROWS_PER_W * _COLS)

    pltpu.sync_copy(in_hbm.at[pl.ds(base, _ROWS_PER_W * _COLS)], row_v)

    def map_body(j, carry):
        base16 = j * (_LANES * 2 * _UNROLL)
        for u in range(2 * _UNROLL):
            x16 = row_v[pl.ds(base16 + u * _LANES, _LANES)]
            key_v[pl.ds(base16 + u * _LANES, _LANES)] = _order_key(
                lax.bitcast_convert_type(x16, jnp.int32))
        return carry

    lax.fori_loop(0, 2 * _VECS // (2 * _UNROLL), map_body, np.int32(0))

    def bit_body(i, state):
        acca, accb, cacca, caccb = state
        bit = np.int32(1) << (np.int32(31) - i)
        ta = acca + bit
        tb = accb + bit
        done = jnp.logical_and(cacca == _KEEP, caccb == _KEEP)
        nj = jnp.where(done, np.int32(0), np.int32(_VECS // _UNROLL))

        def cnt_body(j, cc):
            ca, cb = cc
            base16 = j * (_LANES * _UNROLL)
            for u in range(_UNROLL):
                ka = key_v[pl.ds(base16 + u * _LANES, _LANES)]
                kb = key_v[pl.ds(_COLS + base16 + u * _LANES, _LANES)]
                ca = jnp.where(ka >= ta, ca + np.int32(1), ca)
                cb = jnp.where(kb >= tb, cb + np.int32(1), cb)
            return (ca, cb)

        ca, cb = lax.fori_loop(0, nj, cnt_body, (zeros16, zeros16))
        cnta = _hsum(ca)
        cntb = _hsum(cb)
        hita = jnp.logical_and(cacca != _KEEP, cnta >= _KEEP)
        hitb = jnp.logical_and(caccb != _KEEP, cntb >= _KEEP)
        return (jnp.where(hita, ta, acca),
                jnp.where(hitb, tb, accb),
                jnp.where(hita, cnta, cacca),
                jnp.where(hitb, cntb, caccb))

    thra, thrb, _, _ = lax.fori_loop(
        0, 32, bit_body,
        (_INT_MIN, _INT_MIN, np.int32(_COLS), np.int32(_COLS)))

    def mask_body(j, carry):
        base16 = j * (_LANES * _UNROLL)
        for u in range(_UNROLL):
            for half, thr in ((0, thra), (_COLS, thrb)):
                x16 = row_v[pl.ds(half + base16 + u * _LANES, _LANES)]
                k16 = key_v[pl.ds(half + base16 + u * _LANES, _LANES)]
                row_v[pl.ds(half + base16 + u * _LANES, _LANES)] = jnp.where(
                    k16 >= thr, x16, np.float32(0.0))
        return carry

    lax.fori_loop(0, _VECS // _UNROLL, mask_body, np.int32(0))

    pltpu.sync_copy(row_v, out_hbm.at[pl.ds(base, _ROWS_PER_W * _COLS)])


@functools.partial(jax.jit, static_argnums=())
def _kwta(flat):
    mesh = plsc.VectorSubcoreMesh(core_axis_name="c", subcore_axis_name="s")
    fn = functools.partial(
        pl.kernel,
        mesh=mesh,
        out_type=jax.ShapeDtypeStruct((_ROWS * _COLS,), jnp.float32),
        scratch_types=[
            pltpu.VMEM((_ROWS_PER_W * _COLS,), jnp.float32),
            pltpu.VMEM((_ROWS_PER_W * _COLS,), jnp.int32),
        ],
    )(_kwta_body)
    return fn(flat)


def kernel(inputs):
    out_flat = _kwta(inputs.reshape(-1))
    return out_flat.reshape(inputs.shape)
